# baseline (speedup 1.0000x reference)
import jax
import jax.numpy as jnp
from jax import lax
from jax.experimental import pallas as pl

_TILE = 512


def _fp_kernel(xyz1_ref, xyz2t_ref, points1_ref, points2_ref,
               W1a_ref, W1b_ref, b1_ref, g1_ref, beta1_ref,
               W2_ref, b2_ref, g2_ref, beta2_ref,
               out_ref):
    T = xyz1_ref.shape[1]
    N2 = xyz2t_ref.shape[2]

    x1 = xyz1_ref[0]
    x2t = xyz2t_ref[0]

    u0, u1, u2 = x1[:, 0:1], x1[:, 1:2], x1[:, 2:3]
    v0, v1, v2 = x2t[0:1, :], x2t[1:2, :], x2t[2:3, :]
    ab = jnp.dot(x1, x2t, preferred_element_type=jnp.float32)
    a2 = u0 * u0 + u1 * u1 + u2 * u2
    b2 = v0 * v0 + v1 * v1 + v2 * v2
    d2 = a2 + b2 - 2.0 * ab
    dist = jnp.sqrt(jnp.maximum(d2, 0.0))

    iota = lax.broadcasted_iota(jnp.int32, (T, N2), 1)
    BIG = jnp.float32(3.0e38)
    cur = dist
    ms, ohs = [], []
    for k in range(3):
        m = jnp.min(cur, axis=1, keepdims=True)
        sel_iota = jnp.where(cur == m, iota, N2)
        idxk = jnp.min(sel_iota, axis=1, keepdims=True)
        onehot = (sel_iota == idxk).astype(jnp.float32)
        ms.append(m)
        ohs.append(onehot)
        if k < 2:
            cur = cur + onehot * BIG
    rec = [1.0 / jnp.maximum(m, 1e-10) for m in ms]
    norm = rec[0] + rec[1] + rec[2]
    S = (ohs[0] * (rec[0] / norm) + ohs[1] * (rec[1] / norm)
         + ohs[2] * (rec[2] / norm))

    interp = jnp.dot(S, points2_ref[0],
                     preferred_element_type=jnp.float32)

    inv_std = 1.0 / jnp.sqrt(jnp.float32(1.0 + 1e-5))
    x = (jnp.dot(interp, W1a_ref[...], preferred_element_type=jnp.float32)
         + jnp.dot(points1_ref[0], W1b_ref[...],
                   preferred_element_type=jnp.float32))
    x = x + b1_ref[0]
    x = g1_ref[0] * (x * inv_std) + beta1_ref[0]
    x = jnp.maximum(x, 0.0)

    x = jnp.dot(x, W2_ref[...], preferred_element_type=jnp.float32)
    x = x + b2_ref[0]
    x = g2_ref[0] * (x * inv_std) + beta2_ref[0]
    x = jnp.maximum(x, 0.0)
    out_ref[0] = x


def kernel(xyz1, xyz2, points1, points2, W1, b1, g1, beta1, W2, b2, g2, beta2):
    B, N1, _ = xyz1.shape
    _, N2, C2 = points2.shape
    C1 = points1.shape[2]
    Cout = W2.shape[1]
    T = _TILE if N1 % _TILE == 0 else N1

    xyz2t = jnp.transpose(xyz2, (0, 2, 1))
    W1a, W1b = W1[:C2], W1[C2:]
    vecs = [v.reshape(1, -1) for v in (b1, g1, beta1, b2, g2, beta2)]
    b1r, g1r, beta1r, b2r, g2r, beta2r = vecs

    grid = (B, N1 // T)
    const = lambda b, i: (0, 0)
    per_b = lambda b, i: (b, 0, 0)
    per_tile = lambda b, i: (b, i, 0)

    out = pl.pallas_call(
        _fp_kernel,
        grid=grid,
        in_specs=[
            pl.BlockSpec((1, T, 3), per_tile),
            pl.BlockSpec((1, 3, N2), per_b),
            pl.BlockSpec((1, T, C1), per_tile),
            pl.BlockSpec((1, N2, C2), per_b),
            pl.BlockSpec(W1a.shape, const),
            pl.BlockSpec(W1b.shape, const),
            pl.BlockSpec((1, Cout), const),
            pl.BlockSpec((1, Cout), const),
            pl.BlockSpec((1, Cout), const),
            pl.BlockSpec(W2.shape, const),
            pl.BlockSpec((1, Cout), const),
            pl.BlockSpec((1, Cout), const),
            pl.BlockSpec((1, Cout), const),
        ],
        out_specs=pl.BlockSpec((1, T, Cout), per_tile),
        out_shape=jax.ShapeDtypeStruct((B, N1, Cout), jnp.float32),
    )(xyz1, xyz2t, points1, points2,
      W1a, W1b, b1r, g1r, beta1r, W2, b2r, g2r, beta2r)
    return out


# speedup vs baseline: 1.1536x; 1.1536x over previous
import jax
import jax.numpy as jnp
from jax import lax
from jax.experimental import pallas as pl

_TILE = 512


def _fp_kernel(xyz1_ref, xyz2t_ref, points1_ref, points2_ref,
               W1a_ref, W1b_ref, b1_ref, g1_ref, beta1_ref,
               W2_ref, b2_ref, g2_ref, beta2_ref,
               out_ref):
    T = xyz1_ref.shape[1]
    N2 = xyz2t_ref.shape[2]

    x1 = xyz1_ref[0]
    x2t = xyz2t_ref[0]

    u0, u1, u2 = x1[:, 0:1], x1[:, 1:2], x1[:, 2:3]
    v0, v1, v2 = x2t[0:1, :], x2t[1:2, :], x2t[2:3, :]
    ab = jnp.dot(x1, x2t, preferred_element_type=jnp.float32)
    a2 = u0 * u0 + u1 * u1 + u2 * u2
    b2 = v0 * v0 + v1 * v1 + v2 * v2
    d2 = a2 + b2 - 2.0 * ab
    dist = jnp.sqrt(jnp.maximum(d2, 0.0))

    fiota = lax.broadcasted_iota(jnp.int32, (T, N2), 1).astype(jnp.float32)
    fN2 = jnp.float32(N2)
    BIG = jnp.float32(3.0e38)
    cur = dist
    S = jnp.zeros((T, N2), jnp.float32)
    ms = []
    for k in range(3):
        m = jnp.min(cur, axis=1, keepdims=True)
        sel_iota = jnp.where(cur == m, fiota, fN2)
        idxk = jnp.min(sel_iota, axis=1, keepdims=True)
        onehot = sel_iota == idxk
        rec_k = 1.0 / jnp.maximum(m, 1e-10)
        ms.append(rec_k)
        S = jnp.where(onehot, rec_k, S)
        if k < 2:
            cur = jnp.where(onehot, BIG, cur)
    S = S / (ms[0] + ms[1] + ms[2])

    interp = jnp.dot(S, points2_ref[0],
                     preferred_element_type=jnp.float32)

    inv_std = 1.0 / jnp.sqrt(jnp.float32(1.0 + 1e-5))
    x = (jnp.dot(interp, W1a_ref[...], preferred_element_type=jnp.float32)
         + jnp.dot(points1_ref[0], W1b_ref[...],
                   preferred_element_type=jnp.float32))
    x = x + b1_ref[0]
    x = g1_ref[0] * (x * inv_std) + beta1_ref[0]
    x = jnp.maximum(x, 0.0)

    x = jnp.dot(x, W2_ref[...], preferred_element_type=jnp.float32)
    x = x + b2_ref[0]
    x = g2_ref[0] * (x * inv_std) + beta2_ref[0]
    x = jnp.maximum(x, 0.0)
    out_ref[0] = x


def kernel(xyz1, xyz2, points1, points2, W1, b1, g1, beta1, W2, b2, g2, beta2):
    B, N1, _ = xyz1.shape
    _, N2, C2 = points2.shape
    C1 = points1.shape[2]
    Cout = W2.shape[1]
    T = _TILE if N1 % _TILE == 0 else N1

    xyz2t = jnp.transpose(xyz2, (0, 2, 1))
    W1a, W1b = W1[:C2], W1[C2:]
    vecs = [v.reshape(1, -1) for v in (b1, g1, beta1, b2, g2, beta2)]
    b1r, g1r, beta1r, b2r, g2r, beta2r = vecs

    grid = (B, N1 // T)
    const = lambda b, i: (0, 0)
    per_b = lambda b, i: (b, 0, 0)
    per_tile = lambda b, i: (b, i, 0)

    out = pl.pallas_call(
        _fp_kernel,
        grid=grid,
        in_specs=[
            pl.BlockSpec((1, T, 3), per_tile),
            pl.BlockSpec((1, 3, N2), per_b),
            pl.BlockSpec((1, T, C1), per_tile),
            pl.BlockSpec((1, N2, C2), per_b),
            pl.BlockSpec(W1a.shape, const),
            pl.BlockSpec(W1b.shape, const),
            pl.BlockSpec((1, Cout), const),
            pl.BlockSpec((1, Cout), const),
            pl.BlockSpec((1, Cout), const),
            pl.BlockSpec(W2.shape, const),
            pl.BlockSpec((1, Cout), const),
            pl.BlockSpec((1, Cout), const),
            pl.BlockSpec((1, Cout), const),
        ],
        out_specs=pl.BlockSpec((1, T, Cout), per_tile),
        out_shape=jax.ShapeDtypeStruct((B, N1, Cout), jnp.float32),
    )(xyz1, xyz2t, points1, points2,
      W1a, W1b, b1r, g1r, beta1r, W2, b2r, g2r, beta2r)
    return out
